# baseline (speedup 1.0000x reference)
import math

import jax
import jax.numpy as jnp
from jax import lax
from jax.experimental import pallas as pl
from jax.experimental.pallas import tpu as pltpu

_M = 0.2
_S = 30.0
_COS_M = math.cos(_M)
_SIN_M = math.sin(_M)
_TH = math.cos(math.pi - _M)
_MM = math.sin(math.pi - _M) * _M

_BB = 3456


def _body(lab_ref, cos_ref, out_ref):
    i = pl.program_id(0)
    x = cos_ref[...]
    lab = lab_ref[...]
    row = i * _BB + lax.broadcasted_iota(jnp.int32, x.shape, 0)
    m = row == lab
    cv = jnp.sum(jnp.where(m, x, 0.0), axis=0, keepdims=True)
    sine = jnp.sqrt(jnp.clip(1.0 - cv * cv, 0.0, 1.0))
    phi = cv * _COS_M - sine * _SIN_M
    phi = jnp.where(cv - _TH > 0, phi, cv - _MM)
    out_ref[...] = jnp.where(m, _S * phi, _S * x)


def kernel(cosine, label):
    n, v = cosine.shape
    cos_t = cosine.T
    lab2d = label.astype(jnp.int32).reshape(1, n)
    out_t = pl.pallas_call(
        _body,
        grid=(pl.cdiv(v, _BB),),
        in_specs=[
            pl.BlockSpec((1, n), lambda i: (0, 0)),
            pl.BlockSpec((_BB, n), lambda i: (i, 0)),
        ],
        out_specs=pl.BlockSpec((_BB, n), lambda i: (i, 0)),
        out_shape=jax.ShapeDtypeStruct((v, n), jnp.float32),
        compiler_params=pltpu.CompilerParams(
            dimension_semantics=("parallel",),
        ),
    )(lab2d, cos_t)
    return out_t.T
